# baseline (speedup 1.0000x reference)
import functools

import jax
import jax.numpy as jnp
from jax import lax
from jax.experimental import pallas as pl
from jax.experimental.pallas import tpu as pltpu
from jax.experimental.pallas import tpu_sc as plsc

N = 10000
D = 128
NP = 10240
NW = 32
NSUB = 16
CH = 80
CL = 128
CHD = CH
CLD = CL
EP = NW * CH * CL
RPS = NP // NSUB
ZR = 64


def _sc_mesh():
    return plsc.VectorSubcoreMesh(core_axis_name="c", subcore_axis_name="s")


def _deg_sc(src3, dst3, zeros1):
    @functools.partial(
        pl.kernel,
        out_type=[jax.ShapeDtypeStruct((2, NP), jnp.float32),
                  jax.ShapeDtypeStruct((NW, CHD, CLD), jnp.int32)],
        scratch_types=[pltpu.VMEM((CHD, CLD), jnp.int32),
                       pltpu.VMEM((CHD, CLD), jnp.int32),
                       pltpu.VMEM((CHD, CLD), jnp.float32),
                       pltpu.VMEM_SHARED((NP,), jnp.float32)],
        mesh=_sc_mesh(),
    )
    def k(src_h, dst_h, z_h, degp_h, dstp_h, src_v, dst_v, keep_v, deg_acc):
        c = lax.axis_index("c")
        s = lax.axis_index("s")
        w = s * 2 + c
        pltpu.sync_copy(src_h.at[w], src_v)
        pltpu.sync_copy(dst_h.at[w], dst_v)
        pltpu.sync_copy(z_h.at[pl.ds(s * RPS, RPS)],
                        deg_acc.at[pl.ds(s * RPS, RPS)])
        plsc.subcore_barrier()

        def body(j, carry):
            for c8 in range(CLD // 16):
                sl = pl.ds(c8 * 16, 16)
                sv = src_v[j, sl]
                dv = dst_v[j, sl]
                eq = sv == dv
                keep_v[j, sl] = jnp.where(eq, 0.0, 1.0)
                dst_v[j, sl] = jnp.bitwise_or(
                    sv, lax.shift_left(jnp.where(eq, N, dv), 16))
            pltpu.sync_copy(keep_v.at[j], deg_acc.at[src_v.at[j]], add=True)
            return carry

        lax.fori_loop(0, CHD, body, 0)
        plsc.subcore_barrier()
        pltpu.sync_copy(deg_acc.at[pl.ds(s * RPS, RPS)],
                        degp_h.at[c, pl.ds(s * RPS, RPS)])
        pltpu.sync_copy(dst_v, dstp_h.at[w])

    return k(src3, dst3, zeros1)


def _hop_sc(g, dstp3, zeros2):
    @functools.partial(
        pl.kernel,
        out_type=jax.ShapeDtypeStruct((2, NP, D), jnp.float32),
        scratch_types=[pltpu.VMEM((CH, CL), jnp.int32),
                       pltpu.VMEM((2, CL), jnp.int32),
                       pltpu.VMEM((CL,), jnp.int32),
                       pltpu.VMEM((2 * CL, D), jnp.float32),
                       pltpu.VMEM_SHARED((NP, D), jnp.float32),
                       pltpu.SemaphoreType.DMA((2,))],
        mesh=_sc_mesh(),
    )
    def k(g_h, dstp_h, z_h, acc_h, pk_v, sidx, didx, bigbuf, acc, sem):
        c = lax.axis_index("c")
        s = lax.axis_index("s")
        w = s * 2 + c
        pltpu.sync_copy(dstp_h.at[w], pk_v)

        def unpack_src(j, slot):
            for c8 in range(CL // 16):
                sl = pl.ds(c8 * 16, 16)
                sidx[slot, sl] = jnp.bitwise_and(pk_v[j, sl], 0xFFFF)

        def unpack_dst(j):
            for c8 in range(CL // 16):
                sl = pl.ds(c8 * 16, 16)
                didx[sl] = lax.shift_right_logical(pk_v[j, sl], 16)

        def prime(j, carry):
            unpack_src(j, j)
            pltpu.async_copy(g_h.at[sidx.at[j]],
                             bigbuf.at[pl.ds(j * CL, CL)], sem.at[j])
            return carry

        lax.fori_loop(0, 2, prime, 0)
        for z in range(RPS // ZR):
            pltpu.sync_copy(z_h, acc.at[pl.ds(s * RPS + z * ZR, ZR)])
        plsc.subcore_barrier()

        def body(j, carry):
            p = lax.rem(j, 2)
            slot = bigbuf.at[pl.ds(p * CL, CL)]
            pltpu.make_async_copy(g_h.at[sidx.at[p]], slot, sem.at[p]).wait()
            unpack_dst(j)
            pltpu.sync_copy(slot, acc.at[didx], add=True)
            nxt = jnp.minimum(j + 2, CH - 1)
            unpack_src(nxt, p)
            pltpu.async_copy(g_h.at[sidx.at[p]], slot, sem.at[p])
            return carry

        lax.fori_loop(0, CH, body, 0)

        def drain(p, carry):
            pltpu.make_async_copy(g_h.at[sidx.at[p]],
                                  bigbuf.at[pl.ds(p * CL, CL)],
                                  sem.at[p]).wait()
            return carry

        lax.fori_loop(0, 2, drain, 0)
        plsc.subcore_barrier()
        pltpu.sync_copy(acc.at[pl.ds(s * RPS, RPS)],
                        acc_h.at[c, pl.ds(s * RPS, RPS)])

    return k(g, dstp3, zeros2)


def _prep_tc(deg_p, x_pad):
    R = 512
    grid = NP // R

    def body(dp_ref, x_ref, dis_ref, g_ref):
        deg = dp_ref[0] + dp_ref[1] + 1.0
        dis = lax.rsqrt(deg)
        dis_ref[...] = dis
        g_ref[...] = x_ref[...] * dis

    return pl.pallas_call(
        body,
        grid=(grid,),
        in_specs=[pl.BlockSpec((2, R, 1), lambda i: (0, i, 0)),
                  pl.BlockSpec((R, D), lambda i: (i, 0))],
        out_specs=[pl.BlockSpec((R, 1), lambda i: (i, 0)),
                   pl.BlockSpec((R, D), lambda i: (i, 0))],
        out_shape=[jax.ShapeDtypeStruct((NP, 1), jnp.float32),
                   jax.ShapeDtypeStruct((NP, D), jnp.float32)],
    )(deg_p, x_pad)


def _mid_tc(accs, g1, dis):
    R = 512
    grid = NP // R

    def body(a_ref, g_ref, dis_ref, h_ref, g2_ref):
        dis_b = dis_ref[...]
        h1 = (a_ref[0] + a_ref[1] + g_ref[...]) * dis_b
        h_ref[...] = h1
        g2_ref[...] = h1 * dis_b

    return pl.pallas_call(
        body,
        grid=(grid,),
        in_specs=[pl.BlockSpec((2, R, D), lambda i: (0, i, 0)),
                  pl.BlockSpec((R, D), lambda i: (i, 0)),
                  pl.BlockSpec((R, 1), lambda i: (i, 0))],
        out_specs=[pl.BlockSpec((R, D), lambda i: (i, 0)),
                   pl.BlockSpec((R, D), lambda i: (i, 0))],
        out_shape=[jax.ShapeDtypeStruct((NP, D), jnp.float32),
                   jax.ShapeDtypeStruct((NP, D), jnp.float32)],
    )(accs, g1, dis)


def _final_tc(accs, g2, dis, x_pad, h1, W, b2, a2):
    R = 400
    grid = N // R

    def body(a_ref, g_ref, dis_ref, x_ref, h1_ref, w_ref, b_ref, s_ref, o_ref):
        h2 = (a_ref[0] + a_ref[1] + g_ref[...]) * dis_ref[...]
        acc = jnp.dot(x_ref[...], w_ref[0:128, :],
                      preferred_element_type=jnp.float32)
        acc = acc + jnp.dot(h1_ref[...], w_ref[128:256, :],
                            preferred_element_type=jnp.float32)
        acc = acc + jnp.dot(h2, w_ref[256:384, :],
                            preferred_element_type=jnp.float32)
        acc = acc + b_ref[...]
        slope = s_ref[0, 0]
        o_ref[...] = jnp.where(acc > 0, acc, slope * acc)

    return pl.pallas_call(
        body,
        grid=(grid,),
        in_specs=[pl.BlockSpec((2, R, D), lambda i: (0, i, 0)),
                  pl.BlockSpec((R, D), lambda i: (i, 0)),
                  pl.BlockSpec((R, 1), lambda i: (i, 0)),
                  pl.BlockSpec((R, D), lambda i: (i, 0)),
                  pl.BlockSpec((R, D), lambda i: (i, 0)),
                  pl.BlockSpec((3 * D, D), lambda i: (0, 0)),
                  pl.BlockSpec((1, D), lambda i: (0, 0)),
                  pl.BlockSpec((1, 1), lambda i: (0, 0))],
        out_specs=pl.BlockSpec((R, D), lambda i: (i, 0)),
        out_shape=jax.ShapeDtypeStruct((N, D), jnp.float32),
    )(accs, g2, dis, x_pad, h1, W, b2, a2)


def kernel(x, edge_index, W, b, a):
    E = edge_index.shape[1]
    pad = EP - E
    src = edge_index[0]
    dst = edge_index[1]
    zpad = jnp.zeros((pad,), jnp.int32)
    src3d = jnp.concatenate([src, zpad]).reshape(NW, CHD, CLD)
    dst3d = jnp.concatenate([dst, zpad]).reshape(NW, CHD, CLD)
    x_pad = jnp.pad(x, ((0, NP - N), (0, 0)))
    zeros1 = jnp.zeros((NP,), jnp.float32)
    zeros2 = jnp.zeros((ZR, D), jnp.float32)

    deg_p, dstp3d = _deg_sc(src3d, dst3d, zeros1)
    dstp3 = dstp3d.reshape(NW, CH, CL)
    dis, g1 = _prep_tc(deg_p.reshape(2, NP, 1), x_pad)
    acc1 = _hop_sc(g1, dstp3, zeros2)
    h1, g2 = _mid_tc(acc1, g1, dis)
    acc2 = _hop_sc(g2, dstp3, zeros2)
    out = _final_tc(acc2, g2, dis, x_pad, h1, W,
                    b.reshape(1, D), a.reshape(1, 1))
    return out


# speedup vs baseline: 1.0177x; 1.0177x over previous
import functools

import jax
import jax.numpy as jnp
from jax import lax
from jax.experimental import pallas as pl
from jax.experimental.pallas import tpu as pltpu
from jax.experimental.pallas import tpu_sc as plsc

N = 10000
D = 128
NP = 10240
NW = 32
NSUB = 16
CH = 80
CL = 128
CHD = CH
CLD = CL
EP = NW * CH * CL
RPS = NP // NSUB
ZR = 64


def _sc_mesh():
    return plsc.VectorSubcoreMesh(core_axis_name="c", subcore_axis_name="s")


def _deg_sc(src3, dst3, zeros1):
    @functools.partial(
        pl.kernel,
        out_type=[jax.ShapeDtypeStruct((2, NP), jnp.float32),
                  jax.ShapeDtypeStruct((NW, CHD, CLD), jnp.int32)],
        scratch_types=[pltpu.VMEM((CHD, CLD), jnp.int32),
                       pltpu.VMEM((CHD, CLD), jnp.int32),
                       pltpu.VMEM((CHD, CLD), jnp.float32),
                       pltpu.VMEM_SHARED((NP,), jnp.float32)],
        mesh=_sc_mesh(),
    )
    def k(src_h, dst_h, z_h, degp_h, dstp_h, src_v, dst_v, keep_v, deg_acc):
        c = lax.axis_index("c")
        s = lax.axis_index("s")
        w = s * 2 + c
        pltpu.sync_copy(src_h.at[w], src_v)
        pltpu.sync_copy(dst_h.at[w], dst_v)
        pltpu.sync_copy(z_h.at[pl.ds(s * RPS, RPS)],
                        deg_acc.at[pl.ds(s * RPS, RPS)])
        plsc.subcore_barrier()

        def body(j, carry):
            for c8 in range(CLD // 16):
                sl = pl.ds(c8 * 16, 16)
                sv = src_v[j, sl]
                dv = dst_v[j, sl]
                eq = sv == dv
                keep_v[j, sl] = jnp.where(eq, 0.0, 1.0)
                dst_v[j, sl] = jnp.where(eq, N, dv)
            pltpu.sync_copy(keep_v.at[j], deg_acc.at[src_v.at[j]], add=True)
            return carry

        lax.fori_loop(0, CHD, body, 0)
        plsc.subcore_barrier()
        pltpu.sync_copy(deg_acc.at[pl.ds(s * RPS, RPS)],
                        degp_h.at[c, pl.ds(s * RPS, RPS)])
        pltpu.sync_copy(dst_v, dstp_h.at[w])

    return k(src3, dst3, zeros1)


BB = 8
NB = CH // BB


def _hop_sc(g, src3, dstp3, zeros2):
    @functools.partial(
        pl.kernel,
        out_type=jax.ShapeDtypeStruct((2, NP, D), jnp.float32),
        scratch_types=[pltpu.VMEM((CH, CL), jnp.int32),
                       pltpu.VMEM((2 * BB, CL), jnp.int32),
                       pltpu.VMEM((2 * CL, D), jnp.float32),
                       pltpu.VMEM_SHARED((NP, D), jnp.float32),
                       pltpu.SemaphoreType.DMA((2,)),
                       pltpu.SemaphoreType.DMA((2,))],
        mesh=_sc_mesh(),
    )
    def k(g_h, src_h, dstp_h, z_h, acc_h, dst_v, sblk, bigbuf, acc, sem,
          isem):
        c = lax.axis_index("c")
        s = lax.axis_index("s")
        w = s * 2 + c
        pltpu.sync_copy(dstp_h.at[w], dst_v)
        pltpu.sync_copy(src_h.at[w, pl.ds(0, BB)], sblk.at[pl.ds(0, BB)])
        pltpu.async_copy(src_h.at[w, pl.ds(BB, BB)],
                         sblk.at[pl.ds(BB, BB)], isem.at[1])

        for j in range(2):
            pltpu.async_copy(g_h.at[sblk.at[j]],
                             bigbuf.at[pl.ds(j * CL, CL)], sem.at[j])
        for z in range(RPS // ZR):
            pltpu.sync_copy(z_h, acc.at[pl.ds(s * RPS + z * ZR, ZR)])
        plsc.subcore_barrier()

        def blk(b, carry):
            sl = lax.rem(b, 2)
            for r in range(BB):
                p = r % 2
                j = b * BB + r
                slot = bigbuf.at[pl.ds(p * CL, CL)]
                pltpu.make_async_copy(g_h.at[sblk.at[r]], slot,
                                      sem.at[p]).wait()
                pltpu.sync_copy(slot, acc.at[dst_v.at[j]], add=True)
                if r == BB - 2:
                    pltpu.make_async_copy(
                        src_h.at[w, pl.ds(jnp.minimum(b + 1, NB - 1) * BB,
                                          BB)],
                        sblk.at[pl.ds((1 - sl) * BB, BB)],
                        isem.at[1 - sl]).wait()
                nb = b + (r + 2) // BB
                nr = (r + 2) % BB
                row = jnp.where(nb >= NB, BB - 1,
                                jnp.int32(nr)) + lax.rem(jnp.minimum(
                                    nb, NB - 1), 2) * BB
                pltpu.async_copy(g_h.at[sblk.at[row]], slot, sem.at[p])
                if r == BB - 1:
                    pltpu.async_copy(
                        src_h.at[w, pl.ds(jnp.minimum(b + 2, NB - 1) * BB,
                                          BB)],
                        sblk.at[pl.ds(sl * BB, BB)], isem.at[sl])
            return carry

        lax.fori_loop(0, NB, blk, 0)
        pltpu.make_async_copy(src_h.at[w, pl.ds(0, BB)],
                              sblk.at[pl.ds(0, BB)],
                              isem.at[(NB - 1) % 2]).wait()

        def drain(p, carry):
            pltpu.make_async_copy(g_h.at[sblk.at[p]],
                                  bigbuf.at[pl.ds(p * CL, CL)],
                                  sem.at[p]).wait()
            return carry

        lax.fori_loop(0, 2, drain, 0)
        plsc.subcore_barrier()
        pltpu.sync_copy(acc.at[pl.ds(s * RPS, RPS)],
                        acc_h.at[c, pl.ds(s * RPS, RPS)])

    return k(g, src3, dstp3, zeros2)


def _prep_tc(deg_p, x_pad):
    R = 512
    grid = NP // R

    def body(dp_ref, x_ref, dis_ref, g_ref):
        deg = dp_ref[0] + dp_ref[1] + 1.0
        dis = lax.rsqrt(deg)
        dis_ref[...] = dis
        g_ref[...] = x_ref[...] * dis

    return pl.pallas_call(
        body,
        grid=(grid,),
        in_specs=[pl.BlockSpec((2, R, 1), lambda i: (0, i, 0)),
                  pl.BlockSpec((R, D), lambda i: (i, 0))],
        out_specs=[pl.BlockSpec((R, 1), lambda i: (i, 0)),
                   pl.BlockSpec((R, D), lambda i: (i, 0))],
        out_shape=[jax.ShapeDtypeStruct((NP, 1), jnp.float32),
                   jax.ShapeDtypeStruct((NP, D), jnp.float32)],
    )(deg_p, x_pad)


def _mid_tc(accs, g1, dis):
    R = 512
    grid = NP // R

    def body(a_ref, g_ref, dis_ref, h_ref, g2_ref):
        dis_b = dis_ref[...]
        h1 = (a_ref[0] + a_ref[1] + g_ref[...]) * dis_b
        h_ref[...] = h1
        g2_ref[...] = h1 * dis_b

    return pl.pallas_call(
        body,
        grid=(grid,),
        in_specs=[pl.BlockSpec((2, R, D), lambda i: (0, i, 0)),
                  pl.BlockSpec((R, D), lambda i: (i, 0)),
                  pl.BlockSpec((R, 1), lambda i: (i, 0))],
        out_specs=[pl.BlockSpec((R, D), lambda i: (i, 0)),
                   pl.BlockSpec((R, D), lambda i: (i, 0))],
        out_shape=[jax.ShapeDtypeStruct((NP, D), jnp.float32),
                   jax.ShapeDtypeStruct((NP, D), jnp.float32)],
    )(accs, g1, dis)


def _final_tc(accs, g2, dis, x_pad, h1, W, b2, a2):
    R = 400
    grid = N // R

    def body(a_ref, g_ref, dis_ref, x_ref, h1_ref, w_ref, b_ref, s_ref, o_ref):
        h2 = (a_ref[0] + a_ref[1] + g_ref[...]) * dis_ref[...]
        acc = jnp.dot(x_ref[...], w_ref[0:128, :],
                      preferred_element_type=jnp.float32)
        acc = acc + jnp.dot(h1_ref[...], w_ref[128:256, :],
                            preferred_element_type=jnp.float32)
        acc = acc + jnp.dot(h2, w_ref[256:384, :],
                            preferred_element_type=jnp.float32)
        acc = acc + b_ref[...]
        slope = s_ref[0, 0]
        o_ref[...] = jnp.where(acc > 0, acc, slope * acc)

    return pl.pallas_call(
        body,
        grid=(grid,),
        in_specs=[pl.BlockSpec((2, R, D), lambda i: (0, i, 0)),
                  pl.BlockSpec((R, D), lambda i: (i, 0)),
                  pl.BlockSpec((R, 1), lambda i: (i, 0)),
                  pl.BlockSpec((R, D), lambda i: (i, 0)),
                  pl.BlockSpec((R, D), lambda i: (i, 0)),
                  pl.BlockSpec((3 * D, D), lambda i: (0, 0)),
                  pl.BlockSpec((1, D), lambda i: (0, 0)),
                  pl.BlockSpec((1, 1), lambda i: (0, 0))],
        out_specs=pl.BlockSpec((R, D), lambda i: (i, 0)),
        out_shape=jax.ShapeDtypeStruct((N, D), jnp.float32),
    )(accs, g2, dis, x_pad, h1, W, b2, a2)


def kernel(x, edge_index, W, b, a):
    E = edge_index.shape[1]
    pad = EP - E
    src = edge_index[0]
    dst = edge_index[1]
    zpad = jnp.zeros((pad,), jnp.int32)
    src3d = jnp.concatenate([src, zpad]).reshape(NW, CHD, CLD)
    dst3d = jnp.concatenate([dst, zpad]).reshape(NW, CHD, CLD)
    x_pad = jnp.pad(x, ((0, NP - N), (0, 0)))
    zeros1 = jnp.zeros((NP,), jnp.float32)
    zeros2 = jnp.zeros((ZR, D), jnp.float32)

    deg_p, dstp3d = _deg_sc(src3d, dst3d, zeros1)
    src3 = src3d.reshape(NW, CH, CL)
    dstp3 = dstp3d.reshape(NW, CH, CL)
    dis, g1 = _prep_tc(deg_p.reshape(2, NP, 1), x_pad)
    acc1 = _hop_sc(g1, src3, dstp3, zeros2)
    h1, g2 = _mid_tc(acc1, g1, dis)
    acc2 = _hop_sc(g2, src3, dstp3, zeros2)
    out = _final_tc(acc2, g2, dis, x_pad, h1, W,
                    b.reshape(1, D), a.reshape(1, 1))
    return out
